# baseline (speedup 1.0000x reference)
import jax
import jax.numpy as jnp
from jax.experimental import pallas as pl
from jax.experimental.pallas import tpu as pltpu

_B = 16
_N = 1024
_S1 = 204
_S1P = 208
_S2 = 51
_S2P = 56
_R1 = 0.2
_R2 = 0.4
_K = 64
_OUT = 64
_BIG = 3.0e38


def _mm(a, b):
    return jax.lax.dot_general(a, b, (((1,), (0,)), ((), ())),
                               precision=jax.lax.Precision.HIGHEST,
                               preferred_element_type=jnp.float32)


def _mm_rt(a, b):
    return jax.lax.dot_general(a, b, (((1,), (1,)), ((), ())),
                               precision=jax.lax.Precision.HIGHEST,
                               preferred_element_type=jnp.float32)


def _fps(posT, S, sel_ref, iota):
    sel_ref[0:1, :] = jnp.where(iota == 0.0, 1.0, 0.0)
    p0 = posT[:, 0:1]
    diff = posT - p0
    d = (diff[0:1, :] * diff[0:1, :] + diff[1:2, :] * diff[1:2, :]
         + diff[2:3, :] * diff[2:3, :])

    def body(i, d):
        m = jnp.max(d, axis=1, keepdims=True)
        cand = jnp.where(d == m, iota, _BIG)
        idx = jnp.min(cand, axis=1, keepdims=True)
        eq = (iota == idx)
        sel_ref[pl.ds(i, 1), :] = jnp.where(eq, 1.0, 0.0)
        np3 = jnp.sum(jnp.where(eq, posT, 0.0), axis=1, keepdims=True)
        df = posT - np3
        dd = (df[0:1, :] * df[0:1, :] + df[1:2, :] * df[1:2, :]
              + df[2:3, :] * df[2:3, :])
        return jnp.minimum(d, dd)

    jax.lax.fori_loop(1, S, body, d)


def _pair_d2(c_rows, pT):
    acc = None
    for c in range(3):
        df = c_rows[:, c:c + 1] - pT[c:c + 1, :]
        sq = df * df
        acc = sq if acc is None else acc + sq
    return acc


def _topk_gather(d2, r2, feat, gbuf_ref, iota, q):
    def body(k, d2m):
        m = jnp.min(d2m, axis=1, keepdims=True)
        cand = jnp.where(d2m == m, iota, _BIG)
        idx = jnp.min(cand, axis=1, keepdims=True)
        eq = (iota == idx)
        eqf = jnp.where(eq & (m <= r2), 1.0, 0.0)
        g = _mm(eqf, feat)
        gbuf_ref[pl.ds(k, 1), 0:q, :] = g[None]
        return jnp.where(eq, _BIG, d2m)

    jax.lax.fori_loop(0, _K, body, d2)


def _wknn(d2, k, feat, iota):
    Q = d2.shape[0]
    O = jnp.zeros(d2.shape, jnp.float32)
    wsum = jnp.zeros((Q, 1), jnp.float32)

    def body(_, carry):
        d2m, O, wsum = carry
        m = jnp.min(d2m, axis=1, keepdims=True)
        cand = jnp.where(d2m == m, iota, _BIG)
        idx = jnp.min(cand, axis=1, keepdims=True)
        eq = (iota == idx)
        w = 1.0 / jnp.clip(m, 1e-16, None)
        O = O + jnp.where(eq, w, 0.0)
        wsum = wsum + w
        return jnp.where(eq, _BIG, d2m), O, wsum

    _, O, wsum = jax.lax.fori_loop(0, k, body, (d2, O, wsum))
    return _mm(O, feat) / wsum


def _mmd(a, b):
    return jax.lax.dot_general(a, b, (((1,), (0,)), ((), ())),
                               preferred_element_type=jnp.float32)


def _relu(v):
    return jnp.maximum(v, 0.0)


def _body(feat8_ref, pos_ref, posT_ref,
          w11, b11, w12, b12, w13, b13,
          w21, b21, w22, b22, w23, b23,
          w31, b31, w32, b32, w33, b33,
          wf31, bf31, wf32, bf32,
          wf21, bf21, wf22, bf22,
          wf11, bf11, wf12, bf12, wf13, bf13,
          out_ref,
          sel1_ref, sel2_ref, gbuf_ref, g2buf_ref):
    feat8 = feat8_ref[0]
    pos = pos_ref[0]
    posT = posT_ref[0]
    iota_n = jax.lax.broadcasted_iota(jnp.int32, (1, _N), 1).astype(jnp.float32)
    iota_s1 = jax.lax.broadcasted_iota(jnp.int32, (1, _S1), 1).astype(jnp.float32)
    iota_s2 = jax.lax.broadcasted_iota(jnp.int32, (1, _S2), 1).astype(jnp.float32)

    gbuf_ref[:, pl.ds(_S1, _S1P - _S1), :] = jnp.zeros((_K, _S1P - _S1, 8), jnp.float32)
    g2buf_ref[:, pl.ds(_S2, _S2P - _S2), :] = jnp.zeros((_K, _S2P - _S2, 136), jnp.float32)

    _fps(posT, _S1, sel1_ref, iota_n)
    sel1 = sel1_ref[...]
    c1 = _mm(sel1, pos)
    c1T = _mm_rt(posT, sel1)
    d2 = _pair_d2(c1, posT)
    _topk_gather(d2, _R1 * _R1, feat8, gbuf_ref, iota_n, _S1)
    xg = gbuf_ref[...].reshape(_K * _S1P, 8)
    c1p = jnp.concatenate([c1, jnp.zeros((_S1P - _S1, 3), jnp.float32)], axis=0)
    c1b = jnp.broadcast_to(c1p[None], (_K, _S1P, 3)).reshape(_K * _S1P, 3)
    rel = xg[:, 3:6] - c1b
    xin = jnp.concatenate([xg[:, 0:3], rel, xg[:, 6:8]], axis=1)
    h = _relu(_mmd(xin, w11[...]) + b11[...])
    h = _relu(_mmd(h, w12[...]) + b12[...])
    h = _relu(_mmd(h, w13[...]) + b13[...])
    h = h + (xg[:, 6:7] - 1.0) * 1e30
    x1 = jnp.max(h.reshape(_K, _S1P, 128), axis=0)[0:_S1]

    _fps(c1T, _S2, sel2_ref, iota_s1)
    sel2 = sel2_ref[...]
    c2 = _mm(sel2, c1)
    c2T = _mm_rt(c1T, sel2)
    d2b = _pair_d2(c2, c1T)
    feat136 = jnp.concatenate(
        [x1, c1, jnp.ones((_S1, 1), jnp.float32), jnp.zeros((_S1, 4), jnp.float32)], axis=1)
    _topk_gather(d2b, _R2 * _R2, feat136, g2buf_ref, iota_s1, _S2)
    xg2 = g2buf_ref[...].reshape(_K * _S2P, 136)
    c2p = jnp.concatenate([c2, jnp.zeros((_S2P - _S2, 3), jnp.float32)], axis=0)
    c2b = jnp.broadcast_to(c2p[None], (_K, _S2P, 3)).reshape(_K * _S2P, 3)
    rel2 = xg2[:, 128:131] - c2b
    xin2 = jnp.concatenate([xg2[:, 0:128], rel2, xg2[:, 131:136]], axis=1)
    h2 = _relu(_mmd(xin2, w21[...]) + b21[...])
    h2 = _relu(_mmd(h2, w22[...]) + b22[...])
    h2 = _relu(_mmd(h2, w23[...]) + b23[...])
    h2 = h2 + (xg2[:, 131:132] - 1.0) * 1e30
    x2 = jnp.max(h2.reshape(_K, _S2P, 256), axis=0)[0:_S2]

    xin3 = jnp.concatenate([x2, c2, jnp.zeros((_S2, 5), jnp.float32)], axis=1)
    h3 = _relu(_mmd(xin3, w31[...]) + b31[...])
    h3 = _relu(_mmd(h3, w32[...]) + b32[...])
    h3 = _relu(_mmd(h3, w33[...]) + b33[...])
    g = jnp.max(h3, axis=0, keepdims=True)

    xi3 = jnp.broadcast_to(g, (_S2, 1024))
    f = jnp.concatenate([xi3, x2], axis=1)
    f = _relu(_mmd(f, wf31[...]) + bf31[...])
    f3 = _relu(_mmd(f, wf32[...]) + bf32[...])

    d2q = _pair_d2(c1, c2T)
    xi2 = _wknn(d2q, 3, f3, iota_s2)
    f = jnp.concatenate([xi2, x1], axis=1)
    f = _relu(_mmd(f, wf21[...]) + bf21[...])
    f2 = _relu(_mmd(f, wf22[...]) + bf22[...])

    d2q1 = _pair_d2(pos, c1T)
    xi1 = _wknn(d2q1, 3, f2, iota_s1)
    f = jnp.concatenate([xi1, feat8[:, 0:3], jnp.zeros((_N, 5), jnp.float32)], axis=1)
    f = _relu(_mmd(f, wf11[...]) + bf11[...])
    f = _relu(_mmd(f, wf12[...]) + bf12[...])
    f1 = _relu(_mmd(f, wf13[...]) + bf13[...])
    out_ref[...] = f1[None]


def _pad_w(W, rows):
    W = jnp.asarray(W, jnp.float32)
    if W.shape[0] < rows:
        W = jnp.concatenate([W, jnp.zeros((rows - W.shape[0], W.shape[1]), jnp.float32)], axis=0)
    return W


def _full_spec(shape):
    nd = len(shape)
    return pl.BlockSpec(shape, lambda b, _n=nd: (0,) * _n)


def kernel(x, pos, batch, sa1_params, sa2_params, sa3_params, fp3_params, fp2_params, fp1_params):
    x = jnp.asarray(x, jnp.float32).reshape(_B, _N, 3)
    pos = jnp.asarray(pos, jnp.float32).reshape(_B, _N, 3)
    posT = jnp.swapaxes(pos, 1, 2)
    feat8 = jnp.concatenate(
        [x, pos, jnp.ones((_B, _N, 1), jnp.float32), jnp.zeros((_B, _N, 1), jnp.float32)],
        axis=2)

    weights = []

    def add(params, first_pad):
        for i, (W, b) in enumerate(params):
            Wp = _pad_w(W, first_pad) if i == 0 and first_pad else jnp.asarray(W, jnp.float32)
            weights.append(Wp)
            weights.append(jnp.asarray(b, jnp.float32).reshape(1, -1))

    add(sa1_params, 8)
    add(sa2_params, 136)
    add(sa3_params, 264)
    add(fp3_params, None)
    add(fp2_params, None)
    add(fp1_params, 136)

    in_specs = [
        pl.BlockSpec((1, _N, 8), lambda b: (b, 0, 0)),
        pl.BlockSpec((1, _N, 3), lambda b: (b, 0, 0)),
        pl.BlockSpec((1, 3, _N), lambda b: (b, 0, 0)),
    ] + [_full_spec(w.shape) for w in weights]

    out = pl.pallas_call(
        _body,
        grid=(_B,),
        in_specs=in_specs,
        out_specs=pl.BlockSpec((1, _N, _OUT), lambda b: (b, 0, 0)),
        out_shape=jax.ShapeDtypeStruct((_B, _N, _OUT), jnp.float32),
        scratch_shapes=[
            pltpu.VMEM((_S1, _N), jnp.float32),
            pltpu.VMEM((_S2, _S1), jnp.float32),
            pltpu.VMEM((_K, _S1P, 8), jnp.float32),
            pltpu.VMEM((_K, _S2P, 136), jnp.float32),
        ],
        compiler_params=pltpu.CompilerParams(
            dimension_semantics=("parallel",),
        ),
    )(feat8, pos, posT, *weights)
    return out


# speedup vs baseline: 2.0252x; 2.0252x over previous
import jax
import jax.numpy as jnp
from jax.experimental import pallas as pl
from jax.experimental.pallas import tpu as pltpu

_B = 16
_N = 1024
_S1 = 204
_S1P = 208
_S2 = 51
_S2P = 56
_R1 = 0.2
_R2 = 0.4
_K = 64
_OUT = 64
_BIG = 3.0e38


def _mm(a, b):
    return jax.lax.dot_general(a, b, (((1,), (0,)), ((), ())),
                               precision=jax.lax.Precision.HIGHEST,
                               preferred_element_type=jnp.float32)


def _mm_rt(a, b):
    return jax.lax.dot_general(a, b, (((1,), (1,)), ((), ())),
                               precision=jax.lax.Precision.HIGHEST,
                               preferred_element_type=jnp.float32)


def _fps1_body(px_ref, py_ref, pz_ref, sel_ref):
    px = px_ref[...]
    py = py_ref[...]
    pz = pz_ref[...]
    iota = jax.lax.broadcasted_iota(jnp.int32, (_B, _N), 1).astype(jnp.float32)
    sel_ref[:, 0:1, :] = jnp.where(iota == 0.0, 1.0, 0.0)[:, None, :]
    dx = px - px[:, 0:1]
    dy = py - py[:, 0:1]
    dz = pz - pz[:, 0:1]
    d = dx * dx + dy * dy + dz * dz

    def body(i, d):
        m = jnp.max(d, axis=1, keepdims=True)
        cand = jnp.where(d == m, iota, _BIG)
        idx = jnp.min(cand, axis=1, keepdims=True)
        eq = (iota == idx)
        sel_ref[:, pl.ds(i, 1), :] = jnp.where(eq, 1.0, 0.0)[:, None, :]
        npx = jnp.sum(jnp.where(eq, px, 0.0), axis=1, keepdims=True)
        npy = jnp.sum(jnp.where(eq, py, 0.0), axis=1, keepdims=True)
        npz = jnp.sum(jnp.where(eq, pz, 0.0), axis=1, keepdims=True)
        fx = px - npx
        fy = py - npy
        fz = pz - npz
        dd = fx * fx + fy * fy + fz * fz
        return jnp.minimum(d, dd)

    jax.lax.fori_loop(1, _S1, body, d)


def _fps(posT, S, sel_ref, iota):
    sel_ref[0:1, :] = jnp.where(iota == 0.0, 1.0, 0.0)
    p0 = posT[:, 0:1]
    diff = posT - p0
    d = (diff[0:1, :] * diff[0:1, :] + diff[1:2, :] * diff[1:2, :]
         + diff[2:3, :] * diff[2:3, :])

    def body(i, d):
        m = jnp.max(d, axis=1, keepdims=True)
        cand = jnp.where(d == m, iota, _BIG)
        idx = jnp.min(cand, axis=1, keepdims=True)
        eq = (iota == idx)
        sel_ref[pl.ds(i, 1), :] = jnp.where(eq, 1.0, 0.0)
        np3 = jnp.sum(jnp.where(eq, posT, 0.0), axis=1, keepdims=True)
        df = posT - np3
        dd = (df[0:1, :] * df[0:1, :] + df[1:2, :] * df[1:2, :]
              + df[2:3, :] * df[2:3, :])
        return jnp.minimum(d, dd)

    jax.lax.fori_loop(1, S, body, d)


def _pair_d2(c_rows, pT):
    acc = None
    for c in range(3):
        df = c_rows[:, c:c + 1] - pT[c:c + 1, :]
        sq = df * df
        acc = sq if acc is None else acc + sq
    return acc


def _split_hi_lo(feat):
    hi = feat.astype(jnp.bfloat16).astype(jnp.float32)
    return hi, feat - hi


def _topk_gather(d2, r2, feat, gbuf_ref, iota, q):
    fhi, flo = _split_hi_lo(feat)

    def body(k, d2m):
        m = jnp.min(d2m, axis=1, keepdims=True)
        cand = jnp.where(d2m == m, iota, _BIG)
        idx = jnp.min(cand, axis=1, keepdims=True)
        eq = (iota == idx)
        eqf = jnp.where(eq & (m <= r2), 1.0, 0.0)
        g = _mmd(eqf, fhi) + _mmd(eqf, flo)
        gbuf_ref[pl.ds(k, 1), 0:q, :] = g[None]
        return jnp.where(eq, _BIG, d2m)

    jax.lax.fori_loop(0, _K, body, d2)


def _wknn(d2, k, feat, iota):
    Q = d2.shape[0]
    O = jnp.zeros(d2.shape, jnp.float32)
    wsum = jnp.zeros((Q, 1), jnp.float32)

    def body(_, carry):
        d2m, O, wsum = carry
        m = jnp.min(d2m, axis=1, keepdims=True)
        cand = jnp.where(d2m == m, iota, _BIG)
        idx = jnp.min(cand, axis=1, keepdims=True)
        eq = (iota == idx)
        w = 1.0 / jnp.clip(m, 1e-16, None)
        O = O + jnp.where(eq, w, 0.0)
        wsum = wsum + w
        return jnp.where(eq, _BIG, d2m), O, wsum

    _, O, wsum = jax.lax.fori_loop(0, k, body, (d2, O, wsum))
    fhi, flo = _split_hi_lo(feat)
    ohi, olo = _split_hi_lo(O)
    acc = _mmd(ohi, fhi) + _mmd(ohi, flo) + _mmd(olo, fhi)
    return acc / wsum


def _mmd(a, b):
    return jax.lax.dot_general(a, b, (((1,), (0,)), ((), ())),
                               preferred_element_type=jnp.float32)


def _relu(v):
    return jnp.maximum(v, 0.0)


def _body(feat8_ref, pos_ref, posT_ref, sel1_ref,
          w11, b11, w12, b12, w13, b13,
          w21, b21, w22, b22, w23, b23,
          w31, b31, w32, b32, w33, b33,
          wf31, bf31, wf32, bf32,
          wf21, bf21, wf22, bf22,
          wf11, bf11, wf12, bf12, wf13, bf13,
          out_ref,
          sel2_ref, gbuf_ref, g2buf_ref):
    feat8 = feat8_ref[0]
    pos = pos_ref[0]
    posT = posT_ref[0]
    iota_n = jax.lax.broadcasted_iota(jnp.int32, (1, _N), 1).astype(jnp.float32)
    iota_s1 = jax.lax.broadcasted_iota(jnp.int32, (1, _S1), 1).astype(jnp.float32)
    iota_s2 = jax.lax.broadcasted_iota(jnp.int32, (1, _S2), 1).astype(jnp.float32)

    gbuf_ref[:, pl.ds(_S1, _S1P - _S1), :] = jnp.zeros((_K, _S1P - _S1, 8), jnp.float32)
    g2buf_ref[:, pl.ds(_S2, _S2P - _S2), :] = jnp.zeros((_K, _S2P - _S2, 136), jnp.float32)

    sel1 = sel1_ref[0]
    c1 = _mm(sel1, pos)
    c1T = _mm_rt(posT, sel1)
    d2 = _pair_d2(c1, posT)
    _topk_gather(d2, _R1 * _R1, feat8, gbuf_ref, iota_n, _S1)
    xg = gbuf_ref[...].reshape(_K * _S1P, 8)
    c1p = jnp.concatenate([c1, jnp.zeros((_S1P - _S1, 3), jnp.float32)], axis=0)
    c1b = jnp.broadcast_to(c1p[None], (_K, _S1P, 3)).reshape(_K * _S1P, 3)
    rel = xg[:, 3:6] - c1b
    xin = jnp.concatenate([xg[:, 0:3], rel, xg[:, 6:8]], axis=1)
    h = _relu(_mmd(xin, w11[...]) + b11[...])
    h = _relu(_mmd(h, w12[...]) + b12[...])
    h = _relu(_mmd(h, w13[...]) + b13[...])
    h = h + (xg[:, 6:7] - 1.0) * 1e30
    x1 = jnp.max(h.reshape(_K, _S1P, 128), axis=0)[0:_S1]

    _fps(c1T, _S2, sel2_ref, iota_s1)
    sel2 = sel2_ref[...]
    c2 = _mm(sel2, c1)
    c2T = _mm_rt(c1T, sel2)
    d2b = _pair_d2(c2, c1T)
    feat136 = jnp.concatenate(
        [x1, c1, jnp.ones((_S1, 1), jnp.float32), jnp.zeros((_S1, 4), jnp.float32)], axis=1)
    _topk_gather(d2b, _R2 * _R2, feat136, g2buf_ref, iota_s1, _S2)
    xg2 = g2buf_ref[...].reshape(_K * _S2P, 136)
    c2p = jnp.concatenate([c2, jnp.zeros((_S2P - _S2, 3), jnp.float32)], axis=0)
    c2b = jnp.broadcast_to(c2p[None], (_K, _S2P, 3)).reshape(_K * _S2P, 3)
    rel2 = xg2[:, 128:131] - c2b
    xin2 = jnp.concatenate([xg2[:, 0:128], rel2, xg2[:, 131:136]], axis=1)
    h2 = _relu(_mmd(xin2, w21[...]) + b21[...])
    h2 = _relu(_mmd(h2, w22[...]) + b22[...])
    h2 = _relu(_mmd(h2, w23[...]) + b23[...])
    h2 = h2 + (xg2[:, 131:132] - 1.0) * 1e30
    x2 = jnp.max(h2.reshape(_K, _S2P, 256), axis=0)[0:_S2]

    xin3 = jnp.concatenate([x2, c2, jnp.zeros((_S2, 5), jnp.float32)], axis=1)
    h3 = _relu(_mmd(xin3, w31[...]) + b31[...])
    h3 = _relu(_mmd(h3, w32[...]) + b32[...])
    h3 = _relu(_mmd(h3, w33[...]) + b33[...])
    g = jnp.max(h3, axis=0, keepdims=True)

    xi3 = jnp.broadcast_to(g, (_S2, 1024))
    f = jnp.concatenate([xi3, x2], axis=1)
    f = _relu(_mmd(f, wf31[...]) + bf31[...])
    f3 = _relu(_mmd(f, wf32[...]) + bf32[...])

    d2q = _pair_d2(c1, c2T)
    xi2 = _wknn(d2q, 3, f3, iota_s2)
    f = jnp.concatenate([xi2, x1], axis=1)
    f = _relu(_mmd(f, wf21[...]) + bf21[...])
    f2 = _relu(_mmd(f, wf22[...]) + bf22[...])

    d2q1 = _pair_d2(pos, c1T)
    xi1 = _wknn(d2q1, 3, f2, iota_s1)
    f = jnp.concatenate([xi1, feat8[:, 0:3], jnp.zeros((_N, 5), jnp.float32)], axis=1)
    f = _relu(_mmd(f, wf11[...]) + bf11[...])
    f = _relu(_mmd(f, wf12[...]) + bf12[...])
    f1 = _relu(_mmd(f, wf13[...]) + bf13[...])
    out_ref[...] = f1[None]


def _pad_w(W, rows):
    W = jnp.asarray(W, jnp.float32)
    if W.shape[0] < rows:
        W = jnp.concatenate([W, jnp.zeros((rows - W.shape[0], W.shape[1]), jnp.float32)], axis=0)
    return W


def _full_spec(shape):
    nd = len(shape)
    return pl.BlockSpec(shape, lambda b, _n=nd: (0,) * _n)


def kernel(x, pos, batch, sa1_params, sa2_params, sa3_params, fp3_params, fp2_params, fp1_params):
    x = jnp.asarray(x, jnp.float32).reshape(_B, _N, 3)
    pos = jnp.asarray(pos, jnp.float32).reshape(_B, _N, 3)
    posT = jnp.swapaxes(pos, 1, 2)
    feat8 = jnp.concatenate(
        [x, pos, jnp.ones((_B, _N, 1), jnp.float32), jnp.zeros((_B, _N, 1), jnp.float32)],
        axis=2)

    weights = []

    def add(params, first_pad):
        for i, (W, b) in enumerate(params):
            Wp = _pad_w(W, first_pad) if i == 0 and first_pad else jnp.asarray(W, jnp.float32)
            weights.append(Wp)
            weights.append(jnp.asarray(b, jnp.float32).reshape(1, -1))

    add(sa1_params, 8)
    add(sa2_params, 136)
    add(sa3_params, 264)
    add(fp3_params, None)
    add(fp2_params, None)
    add(fp1_params, 136)

    sel1_all = pl.pallas_call(
        _fps1_body,
        grid=(1,),
        in_specs=[pl.BlockSpec((_B, _N), lambda i: (0, 0))] * 3,
        out_specs=pl.BlockSpec((_B, _S1, _N), lambda i: (0, 0, 0)),
        out_shape=jax.ShapeDtypeStruct((_B, _S1, _N), jnp.float32),
    )(pos[:, :, 0], pos[:, :, 1], pos[:, :, 2])

    in_specs = [
        pl.BlockSpec((1, _N, 8), lambda b: (b, 0, 0)),
        pl.BlockSpec((1, _N, 3), lambda b: (b, 0, 0)),
        pl.BlockSpec((1, 3, _N), lambda b: (b, 0, 0)),
        pl.BlockSpec((1, _S1, _N), lambda b: (b, 0, 0)),
    ] + [_full_spec(w.shape) for w in weights]

    out = pl.pallas_call(
        _body,
        grid=(_B,),
        in_specs=in_specs,
        out_specs=pl.BlockSpec((1, _N, _OUT), lambda b: (b, 0, 0)),
        out_shape=jax.ShapeDtypeStruct((_B, _N, _OUT), jnp.float32),
        scratch_shapes=[
            pltpu.VMEM((_S2, _S1), jnp.float32),
            pltpu.VMEM((_K, _S1P, 8), jnp.float32),
            pltpu.VMEM((_K, _S2P, 136), jnp.float32),
        ],
        compiler_params=pltpu.CompilerParams(
            dimension_semantics=("parallel",),
        ),
    )(feat8, pos, posT, sel1_all, *weights)
    return out
